# Initial kernel scaffold; baseline (speedup 1.0000x reference)
import jax
import jax.numpy as jnp
from jax.experimental import pallas as pl


def kernel(h, e, edge_index, embed_W, embed_b, edge_embed_W, edge_embed_b, msg_W, msg_b, hupd_W, hupd_b, eupd_W, eupd_b):
    raise NotImplementedError("write your pallas kernel here")


# speedup vs baseline: 1.4516x; 1.4516x over previous
import functools

import jax
import jax.numpy as jnp
from jax import lax
from jax.experimental import pallas as pl
from jax.experimental.pallas import tpu as pltpu
from jax.experimental.pallas import tpu_sc as plsc

N = 10000
E = 160000
H = 512
L = 3
NC = 2
NS = 16
NW = NC * NS

F32 = jnp.float32


def _mesh():
    return plsc.VectorSubcoreMesh(
        core_axis_name="c", subcore_axis_name="s", num_cores=NC, num_subcores=NS
    )


BN = 1000
BE = 1600


def _embed_h_body(x_ref, w_ref, b_ref, o_ref):
    o_ref[...] = (
        jnp.dot(x_ref[...], w_ref[...], preferred_element_type=F32) + b_ref[...]
    )


def _embed_h(x, w, b):
    d = x.shape[1]
    return pl.pallas_call(
        _embed_h_body,
        grid=(N // BN,),
        in_specs=[
            pl.BlockSpec((BN, d), lambda i: (i, 0)),
            pl.BlockSpec((d, H), lambda i: (0, 0)),
            pl.BlockSpec((1, H), lambda i: (0, 0)),
        ],
        out_specs=pl.BlockSpec((BN, H), lambda i: (i, 0)),
        out_shape=jax.ShapeDtypeStruct((N, H), F32),
    )(x, w, b)


def _make_node_mm(widths):
    total = sum(widths)

    def body(x_ref, w_ref, *out_refs):
        r = jnp.dot(x_ref[...], w_ref[...], preferred_element_type=F32)
        off = 0
        for o_ref, w in zip(out_refs, widths):
            o_ref[...] = r[:, off:off + w]
            off += w

    def run(x, wcat):
        return pl.pallas_call(
            body,
            grid=(N // BN,),
            in_specs=[
                pl.BlockSpec((BN, H), lambda i: (i, 0)),
                pl.BlockSpec((H, total), lambda i: (0, 0)),
            ],
            out_specs=[pl.BlockSpec((BN, w), lambda i: (i, 0)) for w in widths],
            out_shape=[jax.ShapeDtypeStruct((N, w), F32) for w in widths],
        )(x, wcat)

    return run


_node_mm_full = _make_node_mm([2 * H, H, H, H])
_node_mm_last = _make_node_mm([H, H, H])


def _edge_mm_first_body(x_ref, ew_ref, eb_ref, w2_ref, o_ref):
    x = jnp.dot(x_ref[...], ew_ref[...], preferred_element_type=F32) + eb_ref[...]
    o_ref[...] = jnp.dot(x, w2_ref[...], preferred_element_type=F32)


def _edge_mm_first(e_raw, eeW, eeb, w2):
    d = e_raw.shape[1]
    wc = w2.shape[1]
    return pl.pallas_call(
        _edge_mm_first_body,
        grid=(E // BE,),
        in_specs=[
            pl.BlockSpec((BE, d), lambda i: (i, 0)),
            pl.BlockSpec((d, H), lambda i: (0, 0)),
            pl.BlockSpec((1, H), lambda i: (0, 0)),
            pl.BlockSpec((H, wc), lambda i: (0, 0)),
        ],
        out_specs=pl.BlockSpec((BE, wc), lambda i: (i, 0)),
        out_shape=jax.ShapeDtypeStruct((E, wc), F32),
    )(e_raw, eeW, eeb, w2)


def _edge_mm_fused_body(ewe_ref, g1e_ref, g2_ref, b_ref, w2_ref, o_ref):
    x = ewe_ref[...] + g1e_ref[...] + g2_ref[...] + b_ref[...]
    o_ref[...] = jnp.dot(x, w2_ref[...], preferred_element_type=F32)


def _edge_mm_fused(ew_prev, g1_prev, g2_prev, b_prev, w2):
    wc = w2.shape[1]
    return pl.pallas_call(
        _edge_mm_fused_body,
        grid=(E // BE,),
        in_specs=[
            pl.BlockSpec((BE, H), lambda i: (i, 1)),
            pl.BlockSpec((BE, H), lambda i: (i, 1)),
            pl.BlockSpec((BE, H), lambda i: (i, 0)),
            pl.BlockSpec((1, H), lambda i: (0, 0)),
            pl.BlockSpec((H, wc), lambda i: (0, 0)),
        ],
        out_specs=pl.BlockSpec((BE, wc), lambda i: (i, 0)),
        out_shape=jax.ShapeDtypeStruct((E, wc), F32),
    )(ew_prev, g1_prev, g2_prev, b_prev, w2)


def _hupd_body(agg_ref, bmat_ref, deg_ref, p4_ref, mb_ref, w_ref, hb_ref, o_ref):
    deg = deg_ref[0, :, 0:1] + deg_ref[1, :, 0:1]
    x = agg_ref[...] + deg * (bmat_ref[...] + mb_ref[...])
    o_ref[...] = (
        p4_ref[...]
        + jnp.dot(x, w_ref[...], preferred_element_type=F32)
        + hb_ref[...]
    )


def _hupd(agg, bmat, deg2, p4, msg_b, wh_m, hupd_b):
    return pl.pallas_call(
        _hupd_body,
        grid=(N // BN,),
        in_specs=[
            pl.BlockSpec((BN, H), lambda i: (i, 0)),
            pl.BlockSpec((BN, H), lambda i: (i, 0)),
            pl.BlockSpec((2, BN, 128), lambda i: (0, i, 0)),
            pl.BlockSpec((BN, H), lambda i: (i, 0)),
            pl.BlockSpec((1, H), lambda i: (0, 0)),
            pl.BlockSpec((H, H), lambda i: (0, 0)),
            pl.BlockSpec((1, H), lambda i: (0, 0)),
        ],
        out_specs=pl.BlockSpec((BN, H), lambda i: (i, 0)),
        out_shape=jax.ShapeDtypeStruct((N, H), F32),
    )(agg, bmat, deg2, p4, msg_b, wh_m, hupd_b)


GC = 40
SC_C = 80
CCH = 128
NP = 10240


def _make_sc_gather(K):
    n_per = E // NW
    n_it = n_per // GC

    @functools.partial(
        pl.kernel,
        out_type=jax.ShapeDtypeStruct((E, K), F32),
        mesh=_mesh(),
        scratch_types=[
            pltpu.VMEM((GC,), jnp.int32),
            pltpu.VMEM((GC, K), F32),
            pltpu.SemaphoreType.DMA,
        ],
    )
    def k(table, idx, out, idx_v, rows_v, sem):
        wid = lax.axis_index("s") * NC + lax.axis_index("c")
        base = wid * n_per

        def body(i, carry):
            off = base + i * GC
            pltpu.sync_copy(idx.at[pl.ds(off, GC)], idx_v)
            pltpu.async_copy(table.at[idx_v], rows_v, sem).wait()
            pltpu.sync_copy(rows_v, out.at[pl.ds(off, GC)])
            return carry

        lax.fori_loop(0, n_it, body, 0)

    return k


_sc_gather_1024 = _make_sc_gather(2 * H)
_sc_gather_512 = _make_sc_gather(H)


def _make_sc_scatter():
    n_per = E // NS
    n_it = n_per // SC_C
    rz = NP // NS
    zb = 128
    n_rows_it = rz // zb

    @functools.partial(
        pl.kernel,
        out_type=jax.ShapeDtypeStruct((NP, H), F32),
        mesh=_mesh(),
        scratch_types=[
            pltpu.VMEM((SC_C,), jnp.int32),
            pltpu.VMEM((SC_C, CCH), F32),
            pltpu.VMEM((zb, CCH), F32),
            pltpu.VMEM_SHARED((NP, CCH), F32),
        ],
    )
    def k(ew, g1, rec, zeros_hbm, out, idx_v, val_v, rbuf_v, acc_sh):
        cid = lax.axis_index("c")
        sid = lax.axis_index("s")
        e_base = sid * n_per
        r_base = sid * rz

        for pass_j in range(H // CCH // NC):
            ccol = (cid + NC * pass_j) * CCH

            pltpu.sync_copy(zeros_hbm, rbuf_v)
            for t in range(n_rows_it):
                pltpu.sync_copy(rbuf_v, acc_sh.at[pl.ds(r_base + t * zb, zb)])
            plsc.subcore_barrier()

            def body(i, carry):
                off = e_base + i * SC_C
                pltpu.sync_copy(rec.at[pl.ds(off, SC_C)], idx_v)
                pltpu.sync_copy(
                    ew.at[pl.ds(off, SC_C), pl.ds(ccol, CCH)], val_v
                )
                pltpu.sync_copy(val_v, acc_sh.at[idx_v], add=True)
                pltpu.sync_copy(
                    g1.at[pl.ds(off, SC_C), pl.ds(ccol, CCH)], val_v
                )
                pltpu.sync_copy(val_v, acc_sh.at[idx_v], add=True)
                return carry

            lax.fori_loop(0, n_it, body, 0)
            plsc.subcore_barrier()

            for t in range(n_rows_it):
                r0 = r_base + t * zb
                pltpu.sync_copy(acc_sh.at[pl.ds(r0, zb)], rbuf_v)
                pltpu.sync_copy(
                    rbuf_v, out.at[pl.ds(r0, zb), pl.ds(ccol, CCH)]
                )
            plsc.subcore_barrier()

    return k


_sc_scatter = _make_sc_scatter()


def _make_sc_deg():
    n_per = E // NW
    n_it = n_per // GC
    rz = NP // NS
    zb = 128
    n_rows_it = rz // zb

    @functools.partial(
        pl.kernel,
        out_type=jax.ShapeDtypeStruct((NC * NP, CCH), F32),
        mesh=_mesh(),
        scratch_types=[
            pltpu.VMEM((GC,), jnp.int32),
            pltpu.VMEM((GC, CCH), F32),
            pltpu.VMEM((zb, CCH), F32),
            pltpu.VMEM_SHARED((NP, CCH), F32),
        ],
    )
    def k(rec, ones_hbm, zeros_hbm, out, idx_v, ones_v, rbuf_v, acc_sh):
        cid = lax.axis_index("c")
        sid = lax.axis_index("s")
        wid = sid * NC + cid
        e_base = wid * n_per
        r_base = sid * rz

        pltpu.sync_copy(zeros_hbm, rbuf_v)
        for t in range(n_rows_it):
            pltpu.sync_copy(rbuf_v, acc_sh.at[pl.ds(r_base + t * zb, zb)])
        pltpu.sync_copy(ones_hbm, ones_v)
        plsc.subcore_barrier()

        def body(i, carry):
            off = e_base + i * GC
            pltpu.sync_copy(rec.at[pl.ds(off, GC)], idx_v)
            pltpu.sync_copy(ones_v, acc_sh.at[idx_v], add=True)
            return carry

        lax.fori_loop(0, n_it, body, 0)
        plsc.subcore_barrier()

        for t in range(n_rows_it):
            r0 = r_base + t * zb
            pltpu.sync_copy(acc_sh.at[pl.ds(r0, zb)], rbuf_v)
            pltpu.sync_copy(rbuf_v, out.at[pl.ds(cid * NP + r0, zb)])

    return k


_sc_deg = _make_sc_deg()


def kernel(h, e, edge_index, embed_W, embed_b, edge_embed_W, edge_embed_b,
           msg_W, msg_b, hupd_W, hupd_b, eupd_W, eupd_b):
    send = edge_index[0]
    rec = edge_index[1]

    zeros_hbm = jnp.zeros((128, CCH), F32)
    ones_hbm = jnp.ones((GC, CCH), F32)

    h_cur = _embed_h(h, embed_W, embed_b.reshape(1, H))
    deg2 = _sc_deg(rec, ones_hbm, zeros_hbm).reshape(NC, NP, CCH)

    ew_prev = None
    g1_prev = None
    g2_prev = None

    for l in range(L):
        last = l == L - 1
        Wm_s, Wm_r, Wm_e = msg_W[l, :H], msg_W[l, H:2 * H], msg_W[l, 2 * H:]
        We_s, We_r, We_e = eupd_W[l, :H], eupd_W[l, H:2 * H], eupd_W[l, 2 * H:]
        Wh_h, Wh_m = hupd_W[l, :H], hupd_W[l, H:]

        if not last:
            wcat = jnp.concatenate([Wm_s, We_s, We_r, Wm_r, Wh_h], axis=1)
            p1, p2, bmat, p4 = _node_mm_full(h_cur, wcat)
            w2 = jnp.concatenate([Wm_e, We_e], axis=1)
            g1 = _sc_gather_1024(p1, send)
            g2 = _sc_gather_512(p2, rec)
        else:
            wcat = jnp.concatenate([Wm_s, Wm_r, Wh_h], axis=1)
            p1, bmat, p4 = _node_mm_last(h_cur, wcat)
            w2 = Wm_e
            g1 = _sc_gather_512(p1, send)
            g2 = None

        if l == 0:
            ew = _edge_mm_first(e, edge_embed_W, edge_embed_b.reshape(1, H), w2)
        else:
            ew = _edge_mm_fused(
                ew_prev, g1_prev, g2_prev, eupd_b[l - 1].reshape(1, H), w2
            )

        agg = _sc_scatter(ew, g1, rec, zeros_hbm)
        h_cur = _hupd(
            agg, bmat, deg2, p4,
            msg_b[l].reshape(1, H), Wh_m, hupd_b[l].reshape(1, H),
        )

        ew_prev, g1_prev, g2_prev = ew, g1, g2

    return h_cur


# speedup vs baseline: 2.9950x; 2.0633x over previous
import functools

import jax
import jax.numpy as jnp
from jax import lax
from jax.experimental import pallas as pl
from jax.experimental.pallas import tpu as pltpu
from jax.experimental.pallas import tpu_sc as plsc

N = 10000
E = 160000
H = 512
L = 3
NC = 2
NS = 16
NW = NC * NS

F32 = jnp.float32


def _mesh():
    return plsc.VectorSubcoreMesh(
        core_axis_name="c", subcore_axis_name="s", num_cores=NC, num_subcores=NS
    )


BN = 1000
BE = 3200
DT = 24


def _embed_h_body(x_ref, w_ref, b_ref, o_ref):
    o_ref[...] = (
        jnp.dot(x_ref[...], w_ref[...], preferred_element_type=F32) + b_ref[...]
    )


def _embed_h(x, w, b):
    d = x.shape[1]
    return pl.pallas_call(
        _embed_h_body,
        grid=(N // BN,),
        in_specs=[
            pl.BlockSpec((BN, d), lambda i: (i, 0)),
            pl.BlockSpec((d, H), lambda i: (0, 0)),
            pl.BlockSpec((1, H), lambda i: (0, 0)),
        ],
        out_specs=pl.BlockSpec((BN, H), lambda i: (i, 0)),
        out_shape=jax.ShapeDtypeStruct((N, H), F32),
    )(x, w, b)


_W_FIRST = [128, 128, 128, 128, H, H, H, H]
_W_MID = _W_FIRST
_W_LAST = [128, 128, 128, 128, H]


def _node_first_body(x_ref, w_ref, *out_refs):
    r = jnp.dot(x_ref[...], w_ref[...], preferred_element_type=F32)
    off = 0
    for o_ref, w in zip(out_refs, _W_FIRST):
        o_ref[...] = r[:, off:off + w]
        off += w


def _node_first(x, wcat):
    return pl.pallas_call(
        _node_first_body,
        grid=(N // BN,),
        in_specs=[
            pl.BlockSpec((BN, H), lambda i: (i, 0)),
            pl.BlockSpec((H, 5 * H), lambda i: (0, 0)),
        ],
        out_specs=[pl.BlockSpec((BN, w), lambda i: (i, 0)) for w in _W_FIRST],
        out_shape=[jax.ShapeDtypeStruct((N, w), F32) for w in _W_FIRST],
    )(x, wcat)


def _node_mid_body(x_ref, u_ref, v_ref, w_ref, wme_ref, wee_ref, *out_refs):
    r = jnp.dot(x_ref[...], w_ref[...], preferred_element_type=F32)
    u = u_ref[...]
    v = v_ref[...]
    qs = r[:, 0:H] + jnp.dot(u, wme_ref[...], preferred_element_type=F32)
    qr = r[:, H:2 * H] + jnp.dot(v, wme_ref[...], preferred_element_type=F32)
    p4 = r[:, 2 * H:3 * H]
    un = r[:, 3 * H:4 * H] + jnp.dot(u, wee_ref[...], preferred_element_type=F32)
    vn = r[:, 4 * H:5 * H] + jnp.dot(v, wee_ref[...], preferred_element_type=F32)
    for j in range(4):
        out_refs[j][...] = qs[:, j * 128:(j + 1) * 128]
    out_refs[4][...] = qr
    out_refs[5][...] = p4
    out_refs[6][...] = un
    out_refs[7][...] = vn


def _node_mid(x, u, v, wcat, wme, wee):
    return pl.pallas_call(
        _node_mid_body,
        grid=(N // BN,),
        in_specs=[
            pl.BlockSpec((BN, H), lambda i: (i, 0)),
            pl.BlockSpec((BN, H), lambda i: (i, 0)),
            pl.BlockSpec((BN, H), lambda i: (i, 0)),
            pl.BlockSpec((H, 5 * H), lambda i: (0, 0)),
            pl.BlockSpec((H, H), lambda i: (0, 0)),
            pl.BlockSpec((H, H), lambda i: (0, 0)),
        ],
        out_specs=[pl.BlockSpec((BN, w), lambda i: (i, 0)) for w in _W_MID],
        out_shape=[jax.ShapeDtypeStruct((N, w), F32) for w in _W_MID],
    )(x, u, v, wcat, wme, wee)


def _node_last_body(x_ref, u_ref, v_ref, w_ref, wme_ref, *out_refs):
    r = jnp.dot(x_ref[...], w_ref[...], preferred_element_type=F32)
    qs = r[:, 0:H] + jnp.dot(
        u_ref[...], wme_ref[...], preferred_element_type=F32
    )
    qr = r[:, H:2 * H] + jnp.dot(
        v_ref[...], wme_ref[...], preferred_element_type=F32
    )
    p4 = r[:, 2 * H:3 * H]
    for j in range(4):
        out_refs[j][...] = qs[:, j * 128:(j + 1) * 128]
    out_refs[4][...] = qr
    out_refs[5][...] = p4


def _node_last(x, u, v, wcat, wme):
    widths = [128, 128, 128, 128, H, H]
    return pl.pallas_call(
        _node_last_body,
        grid=(N // BN,),
        in_specs=[
            pl.BlockSpec((BN, H), lambda i: (i, 0)),
            pl.BlockSpec((BN, H), lambda i: (i, 0)),
            pl.BlockSpec((BN, H), lambda i: (i, 0)),
            pl.BlockSpec((H, 3 * H), lambda i: (0, 0)),
            pl.BlockSpec((H, H), lambda i: (0, 0)),
        ],
        out_specs=[pl.BlockSpec((BN, w), lambda i: (i, 0)) for w in widths],
        out_shape=[jax.ShapeDtypeStruct((N, w), F32) for w in widths],
    )(x, u, v, wcat, wme)


def _composite_body(dt_ref, wme_ref, wee_ref, mb_ref, eb_ref, dm_ref, dn_ref):
    dt = dt_ref[...]
    rows = lax.broadcasted_iota(jnp.int32, (DT, H), 0)
    dm = jnp.dot(dt, wme_ref[...], preferred_element_type=F32)
    dm_ref[...] = dm + jnp.where(rows == 16, mb_ref[...], 0.0)
    dn = jnp.dot(dt, wee_ref[...], preferred_element_type=F32)
    dn_ref[...] = dn + jnp.where(rows == 16, eb_ref[...], 0.0)


def _composite(dt, wme, wee, mb, eb):
    return pl.pallas_call(
        _composite_body,
        grid=(1,),
        in_specs=[
            pl.BlockSpec((DT, H), lambda i: (0, 0)),
            pl.BlockSpec((H, H), lambda i: (0, 0)),
            pl.BlockSpec((H, H), lambda i: (0, 0)),
            pl.BlockSpec((1, H), lambda i: (0, 0)),
            pl.BlockSpec((1, H), lambda i: (0, 0)),
        ],
        out_specs=[
            pl.BlockSpec((DT, H), lambda i: (0, 0)),
            pl.BlockSpec((DT, H), lambda i: (0, 0)),
        ],
        out_shape=[
            jax.ShapeDtypeStruct((DT, H), F32),
            jax.ShapeDtypeStruct((DT, H), F32),
        ],
    )(dt, wme, wee, mb, eb)


def _er_body(e_ref, dm_ref, o_ref):
    o_ref[...] = (
        jnp.dot(e_ref[...], dm_ref[0:16, :], preferred_element_type=F32)
        + dm_ref[16:17, :]
    )


def _er(e_raw, dm):
    d = e_raw.shape[1]
    return pl.pallas_call(
        _er_body,
        grid=(E // BE,),
        in_specs=[
            pl.BlockSpec((BE, d), lambda i: (i, 0)),
            pl.BlockSpec((DT, H), lambda i: (0, 0)),
        ],
        out_specs=pl.BlockSpec((BE, H), lambda i: (i, 0)),
        out_shape=jax.ShapeDtypeStruct((E, H), F32),
    )(e_raw, dm)


def _hupd_body(agg_ref, qr_ref, deg_ref, p4_ref, w_ref, hb_ref, o_ref):
    deg = deg_ref[0, :, 0:1] + deg_ref[1, :, 0:1]
    x = agg_ref[...] + deg * qr_ref[...]
    o_ref[...] = (
        p4_ref[...]
        + jnp.dot(x, w_ref[...], preferred_element_type=F32)
        + hb_ref[...]
    )


def _hupd(agg, qr, deg2, p4, wh_m, hupd_b):
    return pl.pallas_call(
        _hupd_body,
        grid=(N // BN,),
        in_specs=[
            pl.BlockSpec((BN, H), lambda i: (i, 0)),
            pl.BlockSpec((BN, H), lambda i: (i, 0)),
            pl.BlockSpec((2, BN, 128), lambda i: (0, i, 0)),
            pl.BlockSpec((BN, H), lambda i: (i, 0)),
            pl.BlockSpec((H, H), lambda i: (0, 0)),
            pl.BlockSpec((1, H), lambda i: (0, 0)),
        ],
        out_specs=pl.BlockSpec((BN, H), lambda i: (i, 0)),
        out_shape=jax.ShapeDtypeStruct((N, H), F32),
    )(agg, qr, deg2, p4, wh_m, hupd_b)


GC = 40
SC_C = 80
CCH = 128
NP = 10240


def _make_sc_scatter():
    n_per = E // NS
    n_it = n_per // SC_C
    rz = NP // NS
    zb = 128
    n_rows_it = rz // zb

    @functools.partial(
        pl.kernel,
        out_type=jax.ShapeDtypeStruct((NP, H), F32),
        mesh=_mesh(),
        scratch_types=[
            pltpu.VMEM((SC_C,), jnp.int32),
            pltpu.VMEM((SC_C,), jnp.int32),
            pltpu.VMEM((SC_C, CCH), F32),
            pltpu.VMEM((SC_C, CCH), F32),
            pltpu.VMEM((zb, CCH), F32),
            pltpu.VMEM_SHARED((NP, CCH), F32),
            pltpu.SemaphoreType.DMA,
        ],
    )
    def k(p0, p1, p2, p3, er, send, rec, zeros_hbm, out,
          sidx_v, ridx_v, val_v, gat_v, rbuf_v, acc_sh, sem):
        cid = lax.axis_index("c")
        sid = lax.axis_index("s")
        e_base = sid * n_per
        r_base = sid * rz
        tables = (p0, p1, p2, p3)

        for chunk_k in range(4):
            @pl.when(cid == chunk_k // 2)
            def _pass(chunk_k=chunk_k):
                ccol = chunk_k * CCH
                table = tables[chunk_k]

                pltpu.sync_copy(zeros_hbm, rbuf_v)
                for t in range(n_rows_it):
                    pltpu.sync_copy(
                        rbuf_v, acc_sh.at[pl.ds(r_base + t * zb, zb)]
                    )
                plsc.subcore_barrier()

                def body(i, carry):
                    off = e_base + i * SC_C
                    pltpu.sync_copy(send.at[pl.ds(off, SC_C)], sidx_v)
                    pltpu.async_copy(table.at[sidx_v], gat_v, sem)
                    pltpu.sync_copy(rec.at[pl.ds(off, SC_C)], ridx_v)
                    pltpu.sync_copy(
                        er.at[pl.ds(off, SC_C), pl.ds(ccol, CCH)], val_v
                    )
                    pltpu.sync_copy(val_v, acc_sh.at[ridx_v], add=True)
                    pltpu.make_async_copy(table.at[sidx_v], gat_v, sem).wait()
                    pltpu.sync_copy(gat_v, acc_sh.at[ridx_v], add=True)
                    return carry

                lax.fori_loop(0, n_it, body, 0)
                plsc.subcore_barrier()

                for t in range(n_rows_it):
                    r0 = r_base + t * zb
                    pltpu.sync_copy(acc_sh.at[pl.ds(r0, zb)], rbuf_v)
                    pltpu.sync_copy(
                        rbuf_v, out.at[pl.ds(r0, zb), pl.ds(ccol, CCH)]
                    )
                plsc.subcore_barrier()

    return k


_sc_scatter = _make_sc_scatter()


def _make_sc_deg():
    n_per = E // NW
    n_it = n_per // GC
    rz = NP // NS
    zb = 128
    n_rows_it = rz // zb

    @functools.partial(
        pl.kernel,
        out_type=jax.ShapeDtypeStruct((NC * NP, CCH), F32),
        mesh=_mesh(),
        scratch_types=[
            pltpu.VMEM((GC,), jnp.int32),
            pltpu.VMEM((GC, CCH), F32),
            pltpu.VMEM((zb, CCH), F32),
            pltpu.VMEM_SHARED((NP, CCH), F32),
        ],
    )
    def k(rec, ones_hbm, zeros_hbm, out, idx_v, ones_v, rbuf_v, acc_sh):
        cid = lax.axis_index("c")
        sid = lax.axis_index("s")
        wid = sid * NC + cid
        e_base = wid * n_per
        r_base = sid * rz

        pltpu.sync_copy(zeros_hbm, rbuf_v)
        for t in range(n_rows_it):
            pltpu.sync_copy(rbuf_v, acc_sh.at[pl.ds(r_base + t * zb, zb)])
        pltpu.sync_copy(ones_hbm, ones_v)
        plsc.subcore_barrier()

        def body(i, carry):
            off = e_base + i * GC
            pltpu.sync_copy(rec.at[pl.ds(off, GC)], idx_v)
            pltpu.sync_copy(ones_v, acc_sh.at[idx_v], add=True)
            return carry

        lax.fori_loop(0, n_it, body, 0)
        plsc.subcore_barrier()

        for t in range(n_rows_it):
            r0 = r_base + t * zb
            pltpu.sync_copy(acc_sh.at[pl.ds(r0, zb)], rbuf_v)
            pltpu.sync_copy(rbuf_v, out.at[pl.ds(cid * NP + r0, zb)])

    return k


_sc_deg = _make_sc_deg()


def kernel(h, e, edge_index, embed_W, embed_b, edge_embed_W, edge_embed_b,
           msg_W, msg_b, hupd_W, hupd_b, eupd_W, eupd_b):
    send = edge_index[0]
    rec = edge_index[1]

    zeros_hbm = jnp.zeros((128, CCH), F32)
    ones_hbm = jnp.ones((GC, CCH), F32)

    h_cur = _embed_h(h, embed_W, embed_b.reshape(1, H))
    deg2 = _sc_deg(rec, ones_hbm, zeros_hbm).reshape(NC, NP, CCH)

    dt = jnp.zeros((DT, H), F32)
    dt = dt.at[0:16, :].set(edge_embed_W).at[16, :].set(edge_embed_b)

    u = v = None
    for l in range(L):
        Wm_s, Wm_r, Wm_e = msg_W[l, :H], msg_W[l, H:2 * H], msg_W[l, 2 * H:]
        We_s, We_r, We_e = eupd_W[l, :H], eupd_W[l, H:2 * H], eupd_W[l, 2 * H:]
        Wh_h, Wh_m = hupd_W[l, :H], hupd_W[l, H:]

        if l == 0:
            wcat = jnp.concatenate([Wm_s, Wm_r, Wh_h, We_s, We_r], axis=1)
            pm0, pm1, pm2, pm3, qr, p4, un, vn = _node_first(h_cur, wcat)
        elif l < L - 1:
            wcat = jnp.concatenate([Wm_s, Wm_r, Wh_h, We_s, We_r], axis=1)
            pm0, pm1, pm2, pm3, qr, p4, un, vn = _node_mid(
                h_cur, u, v, wcat, Wm_e, We_e
            )
        else:
            wcat = jnp.concatenate([Wm_s, Wm_r, Wh_h], axis=1)
            pm0, pm1, pm2, pm3, qr, p4 = _node_last(h_cur, u, v, wcat, Wm_e)
            un = vn = None

        dm, dt = _composite(
            dt, Wm_e, We_e, msg_b[l].reshape(1, H), eupd_b[l].reshape(1, H)
        )
        er = _er(e, dm)

        agg = _sc_scatter(pm0, pm1, pm2, pm3, er, send, rec, zeros_hbm)
        h_cur = _hupd(agg, qr, deg2, p4, Wh_m, hupd_b[l].reshape(1, H))
        u, v = un, vn

    return h_cur


# speedup vs baseline: 4.0253x; 1.3440x over previous
import functools

import jax
import jax.numpy as jnp
from jax import lax
from jax.experimental import pallas as pl
from jax.experimental.pallas import tpu as pltpu
from jax.experimental.pallas import tpu_sc as plsc

N = 10000
E = 160000
H = 512
L = 3
NC = 2
NS = 16
NW = NC * NS

F32 = jnp.float32


def _mesh():
    return plsc.VectorSubcoreMesh(
        core_axis_name="c", subcore_axis_name="s", num_cores=NC, num_subcores=NS
    )


BN = 1000
BE = 3200
DT = 24


def _embed_h_body(x_ref, w_ref, b_ref, o_ref):
    o_ref[...] = (
        jnp.dot(x_ref[...], w_ref[...], preferred_element_type=F32) + b_ref[...]
    )


def _embed_h(x, w, b):
    d = x.shape[1]
    return pl.pallas_call(
        _embed_h_body,
        grid=(N // BN,),
        in_specs=[
            pl.BlockSpec((BN, d), lambda i: (i, 0)),
            pl.BlockSpec((d, H), lambda i: (0, 0)),
            pl.BlockSpec((1, H), lambda i: (0, 0)),
        ],
        out_specs=pl.BlockSpec((BN, H), lambda i: (i, 0)),
        out_shape=jax.ShapeDtypeStruct((N, H), F32),
    )(x, w, b)


_W_FIRST = [128, 128, 128, 128, H, H, H, H]
_W_MID = _W_FIRST
_W_LAST = [128, 128, 128, 128, H]


def _node_first_body(x_ref, w_ref, *out_refs):
    r = jnp.dot(x_ref[...], w_ref[...], preferred_element_type=F32)
    off = 0
    for o_ref, w in zip(out_refs, _W_FIRST):
        o_ref[...] = r[:, off:off + w]
        off += w


def _node_first(x, wcat):
    return pl.pallas_call(
        _node_first_body,
        grid=(N // BN,),
        in_specs=[
            pl.BlockSpec((BN, H), lambda i: (i, 0)),
            pl.BlockSpec((H, 5 * H), lambda i: (0, 0)),
        ],
        out_specs=[pl.BlockSpec((BN, w), lambda i: (i, 0)) for w in _W_FIRST],
        out_shape=[jax.ShapeDtypeStruct((N, w), F32) for w in _W_FIRST],
    )(x, wcat)


def _node_mid_body(x_ref, u_ref, v_ref, w_ref, wme_ref, wee_ref, *out_refs):
    r = jnp.dot(x_ref[...], w_ref[...], preferred_element_type=F32)
    u = u_ref[...]
    v = v_ref[...]
    qs = r[:, 0:H] + jnp.dot(u, wme_ref[...], preferred_element_type=F32)
    qr = r[:, H:2 * H] + jnp.dot(v, wme_ref[...], preferred_element_type=F32)
    p4 = r[:, 2 * H:3 * H]
    un = r[:, 3 * H:4 * H] + jnp.dot(u, wee_ref[...], preferred_element_type=F32)
    vn = r[:, 4 * H:5 * H] + jnp.dot(v, wee_ref[...], preferred_element_type=F32)
    for j in range(4):
        out_refs[j][...] = qs[:, j * 128:(j + 1) * 128]
    out_refs[4][...] = qr
    out_refs[5][...] = p4
    out_refs[6][...] = un
    out_refs[7][...] = vn


def _node_mid(x, u, v, wcat, wme, wee):
    return pl.pallas_call(
        _node_mid_body,
        grid=(N // BN,),
        in_specs=[
            pl.BlockSpec((BN, H), lambda i: (i, 0)),
            pl.BlockSpec((BN, H), lambda i: (i, 0)),
            pl.BlockSpec((BN, H), lambda i: (i, 0)),
            pl.BlockSpec((H, 5 * H), lambda i: (0, 0)),
            pl.BlockSpec((H, H), lambda i: (0, 0)),
            pl.BlockSpec((H, H), lambda i: (0, 0)),
        ],
        out_specs=[pl.BlockSpec((BN, w), lambda i: (i, 0)) for w in _W_MID],
        out_shape=[jax.ShapeDtypeStruct((N, w), F32) for w in _W_MID],
    )(x, u, v, wcat, wme, wee)


def _node_last_body(x_ref, u_ref, v_ref, w_ref, wme_ref, *out_refs):
    r = jnp.dot(x_ref[...], w_ref[...], preferred_element_type=F32)
    qs = r[:, 0:H] + jnp.dot(
        u_ref[...], wme_ref[...], preferred_element_type=F32
    )
    qr = r[:, H:2 * H] + jnp.dot(
        v_ref[...], wme_ref[...], preferred_element_type=F32
    )
    p4 = r[:, 2 * H:3 * H]
    for j in range(4):
        out_refs[j][...] = qs[:, j * 128:(j + 1) * 128]
    out_refs[4][...] = qr
    out_refs[5][...] = p4


def _node_last(x, u, v, wcat, wme):
    widths = [128, 128, 128, 128, H, H]
    return pl.pallas_call(
        _node_last_body,
        grid=(N // BN,),
        in_specs=[
            pl.BlockSpec((BN, H), lambda i: (i, 0)),
            pl.BlockSpec((BN, H), lambda i: (i, 0)),
            pl.BlockSpec((BN, H), lambda i: (i, 0)),
            pl.BlockSpec((H, 3 * H), lambda i: (0, 0)),
            pl.BlockSpec((H, H), lambda i: (0, 0)),
        ],
        out_specs=[pl.BlockSpec((BN, w), lambda i: (i, 0)) for w in widths],
        out_shape=[jax.ShapeDtypeStruct((N, w), F32) for w in widths],
    )(x, u, v, wcat, wme)


def _composite_body(dt_ref, wme_ref, wee_ref, mb_ref, eb_ref, dm_ref, dn_ref):
    dt = dt_ref[...]
    rows = lax.broadcasted_iota(jnp.int32, (DT, H), 0)
    dm = jnp.dot(dt, wme_ref[...], preferred_element_type=F32)
    dm_ref[...] = dm + jnp.where(rows == 16, mb_ref[...], 0.0)
    dn = jnp.dot(dt, wee_ref[...], preferred_element_type=F32)
    dn_ref[...] = dn + jnp.where(rows == 16, eb_ref[...], 0.0)


def _composite(dt, wme, wee, mb, eb):
    return pl.pallas_call(
        _composite_body,
        grid=(1,),
        in_specs=[
            pl.BlockSpec((DT, H), lambda i: (0, 0)),
            pl.BlockSpec((H, H), lambda i: (0, 0)),
            pl.BlockSpec((H, H), lambda i: (0, 0)),
            pl.BlockSpec((1, H), lambda i: (0, 0)),
            pl.BlockSpec((1, H), lambda i: (0, 0)),
        ],
        out_specs=[
            pl.BlockSpec((DT, H), lambda i: (0, 0)),
            pl.BlockSpec((DT, H), lambda i: (0, 0)),
        ],
        out_shape=[
            jax.ShapeDtypeStruct((DT, H), F32),
            jax.ShapeDtypeStruct((DT, H), F32),
        ],
    )(dt, wme, wee, mb, eb)


def _er_body(e_ref, dm_ref, o_ref):
    o_ref[...] = (
        jnp.dot(e_ref[...], dm_ref[0:16, :], preferred_element_type=F32)
        + dm_ref[16:17, :]
    )


def _er(e_raw, dm):
    d = e_raw.shape[1]
    return pl.pallas_call(
        _er_body,
        grid=(E // BE,),
        in_specs=[
            pl.BlockSpec((BE, d), lambda i: (i, 0)),
            pl.BlockSpec((DT, H), lambda i: (0, 0)),
        ],
        out_specs=pl.BlockSpec((BE, H), lambda i: (i, 0)),
        out_shape=jax.ShapeDtypeStruct((E, H), F32),
    )(e_raw, dm)


def _hupd_body(agg_ref, qr_ref, deg_ref, p4_ref, w_ref, hb_ref, o_ref):
    deg = deg_ref[0, :, 0:1] + deg_ref[1, :, 0:1]
    x = agg_ref[...] + deg * qr_ref[...]
    o_ref[...] = (
        p4_ref[...]
        + jnp.dot(x, w_ref[...], preferred_element_type=F32)
        + hb_ref[...]
    )


def _hupd(agg, qr, deg2, p4, wh_m, hupd_b):
    return pl.pallas_call(
        _hupd_body,
        grid=(N // BN,),
        in_specs=[
            pl.BlockSpec((BN, H), lambda i: (i, 0)),
            pl.BlockSpec((BN, H), lambda i: (i, 0)),
            pl.BlockSpec((2, BN, 128), lambda i: (0, i, 0)),
            pl.BlockSpec((BN, H), lambda i: (i, 0)),
            pl.BlockSpec((H, H), lambda i: (0, 0)),
            pl.BlockSpec((1, H), lambda i: (0, 0)),
        ],
        out_specs=pl.BlockSpec((BN, H), lambda i: (i, 0)),
        out_shape=jax.ShapeDtypeStruct((N, H), F32),
    )(agg, qr, deg2, p4, wh_m, hupd_b)


GC = 40
SC_C = 80
CCH = 128
NP = 10240


def _make_sc_scatter():
    n_per = E // NS
    n_it = n_per // SC_C
    rz = NP // NS
    zb = 80
    n_rows_it = rz // zb

    @functools.partial(
        pl.kernel,
        out_type=jax.ShapeDtypeStruct((NP, H), F32),
        mesh=_mesh(),
        scratch_types=[
            pltpu.VMEM((SC_C,), jnp.int32),
            pltpu.VMEM((SC_C,), jnp.int32),
            pltpu.VMEM((SC_C, CCH), F32),
            pltpu.VMEM((SC_C, CCH), F32),
            pltpu.VMEM((SC_C,), jnp.int32),
            pltpu.VMEM((SC_C,), jnp.int32),
            pltpu.VMEM((SC_C, CCH), F32),
            pltpu.VMEM((SC_C, CCH), F32),
            pltpu.VMEM_SHARED((NP, CCH), F32),
            pltpu.SemaphoreType.DMA,
            pltpu.SemaphoreType.DMA,
            pltpu.SemaphoreType.DMA,
            pltpu.SemaphoreType.DMA,
        ],
    )
    def k(p0, p1, p2, p3, er, send, rec, zeros_hbm, out,
          sidx_a, ridx_a, val_a, gat_a, sidx_b, ridx_b, val_b, gat_b,
          acc_sh, semg_a, seme_a, semg_b, seme_b):
        cid = lax.axis_index("c")
        sid = lax.axis_index("s")
        e_base = sid * n_per
        r_base = sid * rz
        tables = (p0, p1, p2, p3)
        buf_a = (sidx_a, ridx_a, val_a, gat_a, semg_a, seme_a)
        buf_b = (sidx_b, ridx_b, val_b, gat_b, semg_b, seme_b)

        for chunk_k in range(4):
            @pl.when(cid == chunk_k // 2)
            def _pass(chunk_k=chunk_k):
                ccol = chunk_k * CCH
                table = tables[chunk_k]

                def load(buf, i):
                    sidx, ridx, val, gat, semg, seme = buf
                    off = e_base + i * SC_C
                    pltpu.sync_copy(send.at[pl.ds(off, SC_C)], sidx)
                    pltpu.async_copy(table.at[sidx], gat, semg)
                    pltpu.sync_copy(rec.at[pl.ds(off, SC_C)], ridx)
                    pltpu.async_copy(
                        er.at[pl.ds(off, SC_C), pl.ds(ccol, CCH)], val, seme
                    )

                def drain(buf, i):
                    sidx, ridx, val, gat, semg, seme = buf
                    off = e_base + i * SC_C
                    pltpu.make_async_copy(
                        er.at[pl.ds(off, SC_C), pl.ds(ccol, CCH)], val, seme
                    ).wait()
                    pltpu.sync_copy(val, acc_sh.at[ridx], add=True)
                    pltpu.make_async_copy(table.at[sidx], gat, semg).wait()
                    pltpu.sync_copy(gat, acc_sh.at[ridx], add=True)

                pltpu.sync_copy(zeros_hbm.at[pl.ds(0, zb)], val_a)
                for t in range(n_rows_it):
                    pltpu.sync_copy(
                        val_a, acc_sh.at[pl.ds(r_base + t * zb, zb)]
                    )
                plsc.subcore_barrier()

                load(buf_a, 0)

                def body(i, carry):
                    @pl.when(i % 2 == 0)
                    def _even():
                        @pl.when(i + 1 < n_it)
                        def _pf():
                            load(buf_b, i + 1)
                        drain(buf_a, i)

                    @pl.when(i % 2 == 1)
                    def _odd():
                        @pl.when(i + 1 < n_it)
                        def _pf():
                            load(buf_a, i + 1)
                        drain(buf_b, i)

                    return carry

                lax.fori_loop(0, n_it, body, 0)
                plsc.subcore_barrier()

                for t in range(n_rows_it):
                    r0 = r_base + t * zb
                    pltpu.sync_copy(acc_sh.at[pl.ds(r0, zb)], val_a)
                    pltpu.sync_copy(
                        val_a, out.at[pl.ds(r0, zb), pl.ds(ccol, CCH)]
                    )
                plsc.subcore_barrier()

    return k


_sc_scatter = _make_sc_scatter()


def _make_sc_deg():
    n_per = E // NW
    n_it = n_per // GC
    rz = NP // NS
    zb = 128
    n_rows_it = rz // zb

    @functools.partial(
        pl.kernel,
        out_type=jax.ShapeDtypeStruct((NC * NP, CCH), F32),
        mesh=_mesh(),
        scratch_types=[
            pltpu.VMEM((GC,), jnp.int32),
            pltpu.VMEM((GC, CCH), F32),
            pltpu.VMEM((zb, CCH), F32),
            pltpu.VMEM_SHARED((NP, CCH), F32),
        ],
    )
    def k(rec, ones_hbm, zeros_hbm, out, idx_v, ones_v, rbuf_v, acc_sh):
        cid = lax.axis_index("c")
        sid = lax.axis_index("s")
        wid = sid * NC + cid
        e_base = wid * n_per
        r_base = sid * rz

        pltpu.sync_copy(zeros_hbm, rbuf_v)
        for t in range(n_rows_it):
            pltpu.sync_copy(rbuf_v, acc_sh.at[pl.ds(r_base + t * zb, zb)])
        pltpu.sync_copy(ones_hbm, ones_v)
        plsc.subcore_barrier()

        def body(i, carry):
            off = e_base + i * GC
            pltpu.sync_copy(rec.at[pl.ds(off, GC)], idx_v)
            pltpu.sync_copy(ones_v, acc_sh.at[idx_v], add=True)
            return carry

        lax.fori_loop(0, n_it, body, 0)
        plsc.subcore_barrier()

        for t in range(n_rows_it):
            r0 = r_base + t * zb
            pltpu.sync_copy(acc_sh.at[pl.ds(r0, zb)], rbuf_v)
            pltpu.sync_copy(rbuf_v, out.at[pl.ds(cid * NP + r0, zb)])

    return k


_sc_deg = _make_sc_deg()


def kernel(h, e, edge_index, embed_W, embed_b, edge_embed_W, edge_embed_b,
           msg_W, msg_b, hupd_W, hupd_b, eupd_W, eupd_b):
    send = edge_index[0]
    rec = edge_index[1]

    zeros_hbm = jnp.zeros((128, CCH), F32)
    ones_hbm = jnp.ones((GC, CCH), F32)

    h_cur = _embed_h(h, embed_W, embed_b.reshape(1, H))
    deg2 = _sc_deg(rec, ones_hbm, zeros_hbm).reshape(NC, NP, CCH)

    dt = jnp.zeros((DT, H), F32)
    dt = dt.at[0:16, :].set(edge_embed_W).at[16, :].set(edge_embed_b)

    ers = []
    for l in range(L):
        dm, dt = _composite(
            dt, msg_W[l, 2 * H:], eupd_W[l, 2 * H:],
            msg_b[l].reshape(1, H), eupd_b[l].reshape(1, H),
        )
        ers.append(_er(e, dm))

    u = v = None
    for l in range(L):
        Wm_s, Wm_r, Wm_e = msg_W[l, :H], msg_W[l, H:2 * H], msg_W[l, 2 * H:]
        We_s, We_r, We_e = eupd_W[l, :H], eupd_W[l, H:2 * H], eupd_W[l, 2 * H:]
        Wh_h, Wh_m = hupd_W[l, :H], hupd_W[l, H:]

        if l == 0:
            wcat = jnp.concatenate([Wm_s, Wm_r, Wh_h, We_s, We_r], axis=1)
            pm0, pm1, pm2, pm3, qr, p4, un, vn = _node_first(h_cur, wcat)
        elif l < L - 1:
            wcat = jnp.concatenate([Wm_s, Wm_r, Wh_h, We_s, We_r], axis=1)
            pm0, pm1, pm2, pm3, qr, p4, un, vn = _node_mid(
                h_cur, u, v, wcat, Wm_e, We_e
            )
        else:
            wcat = jnp.concatenate([Wm_s, Wm_r, Wh_h], axis=1)
            pm0, pm1, pm2, pm3, qr, p4 = _node_last(h_cur, u, v, wcat, Wm_e)
            un = vn = None

        agg = _sc_scatter(pm0, pm1, pm2, pm3, ers[l], send, rec, zeros_hbm)
        h_cur = _hupd(agg, qr, deg2, p4, Wh_m, hupd_b[l].reshape(1, H))
        u, v = un, vn

    return h_cur


# speedup vs baseline: 4.1097x; 1.0210x over previous
import functools

import jax
import jax.numpy as jnp
from jax import lax
from jax.experimental import pallas as pl
from jax.experimental.pallas import tpu as pltpu
from jax.experimental.pallas import tpu_sc as plsc

N = 10000
E = 160000
H = 512
L = 3
NC = 2
NS = 16
NW = NC * NS

F32 = jnp.float32


def _mesh():
    return plsc.VectorSubcoreMesh(
        core_axis_name="c", subcore_axis_name="s", num_cores=NC, num_subcores=NS
    )


BN = 1000
BE = 3200
DT = 24


_W_FULL = [128, 128, 128, 128, H, H, H, H]
_W_SHORT = [128, 128, 128, 128, H, H]


def _project(r, u, v, wme_ref, wee_ref, out_refs):
    qs = r[:, 0:H] + jnp.dot(u, wme_ref[...], preferred_element_type=F32)
    qr = r[:, H:2 * H] + jnp.dot(v, wme_ref[...], preferred_element_type=F32)
    for j in range(4):
        out_refs[j][...] = qs[:, j * 128:(j + 1) * 128]
    out_refs[4][...] = qr
    out_refs[5][...] = r[:, 2 * H:3 * H]
    if wee_ref is not None:
        out_refs[6][...] = r[:, 3 * H:4 * H] + jnp.dot(
            u, wee_ref[...], preferred_element_type=F32
        )
        out_refs[7][...] = r[:, 4 * H:5 * H] + jnp.dot(
            v, wee_ref[...], preferred_element_type=F32
        )


def _node_first_body(x_ref, ew_ref, eb_ref, w_ref, *out_refs):
    h0 = jnp.dot(x_ref[...], ew_ref[...], preferred_element_type=F32)
    h0 = h0 + eb_ref[...]
    r = jnp.dot(h0, w_ref[...], preferred_element_type=F32)
    off = 0
    for o_ref, w in zip(out_refs, _W_FULL):
        o_ref[...] = r[:, off:off + w]
        off += w


def _node_first(x, embw, embb, wcat):
    d = x.shape[1]
    return pl.pallas_call(
        _node_first_body,
        grid=(N // BN,),
        in_specs=[
            pl.BlockSpec((BN, d), lambda i: (i, 0)),
            pl.BlockSpec((d, H), lambda i: (0, 0)),
            pl.BlockSpec((1, H), lambda i: (0, 0)),
            pl.BlockSpec((H, 5 * H), lambda i: (0, 0)),
        ],
        out_specs=[pl.BlockSpec((BN, w), lambda i: (i, 0)) for w in _W_FULL],
        out_shape=[jax.ShapeDtypeStruct((N, w), F32) for w in _W_FULL],
    )(x, embw, embb, wcat)


def _hupd_node_body(agg_ref, qr_ref, deg_ref, p4_ref, whm_ref, hb_ref,
                    u_ref, v_ref, w_ref, wme_ref, wee_ref, *out_refs):
    deg = deg_ref[0, :, 0:1] + deg_ref[1, :, 0:1]
    x = agg_ref[...] + deg * qr_ref[...]
    h = (
        p4_ref[...]
        + jnp.dot(x, whm_ref[...], preferred_element_type=F32)
        + hb_ref[...]
    )
    r = jnp.dot(h, w_ref[...], preferred_element_type=F32)
    _project(r, u_ref[...], v_ref[...], wme_ref, wee_ref, out_refs)


def _hupd_node_last_body(agg_ref, qr_ref, deg_ref, p4_ref, whm_ref, hb_ref,
                         u_ref, v_ref, w_ref, wme_ref, *out_refs):
    _hupd_node_body(agg_ref, qr_ref, deg_ref, p4_ref, whm_ref, hb_ref,
                    u_ref, v_ref, w_ref, wme_ref, None, *out_refs)


def _hupd_node(agg, qr, deg2, p4, whm, hb, u, v, wcat, wme, wee=None):
    widths = _W_FULL if wee is not None else _W_SHORT
    nw = wcat.shape[1]
    in_specs = [
        pl.BlockSpec((BN, H), lambda i: (i, 0)),
        pl.BlockSpec((BN, H), lambda i: (i, 0)),
        pl.BlockSpec((2, BN, 128), lambda i: (0, i, 0)),
        pl.BlockSpec((BN, H), lambda i: (i, 0)),
        pl.BlockSpec((H, H), lambda i: (0, 0)),
        pl.BlockSpec((1, H), lambda i: (0, 0)),
        pl.BlockSpec((BN, H), lambda i: (i, 0)),
        pl.BlockSpec((BN, H), lambda i: (i, 0)),
        pl.BlockSpec((H, nw), lambda i: (0, 0)),
        pl.BlockSpec((H, H), lambda i: (0, 0)),
    ]
    args = [agg, qr, deg2, p4, whm, hb, u, v, wcat, wme]
    body = _hupd_node_last_body
    if wee is not None:
        in_specs.append(pl.BlockSpec((H, H), lambda i: (0, 0)))
        args.append(wee)
        body = _hupd_node_body
    return pl.pallas_call(
        body,
        grid=(N // BN,),
        in_specs=in_specs,
        out_specs=[pl.BlockSpec((BN, w), lambda i: (i, 0)) for w in widths],
        out_shape=[jax.ShapeDtypeStruct((N, w), F32) for w in widths],
    )(*args)


def _composite_body(dt_ref, wme_ref, wee_ref, mb_ref, eb_ref, dm_ref, dn_ref):
    dt = dt_ref[...]
    rows = lax.broadcasted_iota(jnp.int32, (DT, H), 0)
    dm = jnp.dot(dt, wme_ref[...], preferred_element_type=F32)
    dm_ref[...] = dm + jnp.where(rows == 16, mb_ref[...], 0.0)
    dn = jnp.dot(dt, wee_ref[...], preferred_element_type=F32)
    dn_ref[...] = dn + jnp.where(rows == 16, eb_ref[...], 0.0)


def _composite(dt, wme, wee, mb, eb):
    return pl.pallas_call(
        _composite_body,
        grid=(1,),
        in_specs=[
            pl.BlockSpec((DT, H), lambda i: (0, 0)),
            pl.BlockSpec((H, H), lambda i: (0, 0)),
            pl.BlockSpec((H, H), lambda i: (0, 0)),
            pl.BlockSpec((1, H), lambda i: (0, 0)),
            pl.BlockSpec((1, H), lambda i: (0, 0)),
        ],
        out_specs=[
            pl.BlockSpec((DT, H), lambda i: (0, 0)),
            pl.BlockSpec((DT, H), lambda i: (0, 0)),
        ],
        out_shape=[
            jax.ShapeDtypeStruct((DT, H), F32),
            jax.ShapeDtypeStruct((DT, H), F32),
        ],
    )(dt, wme, wee, mb, eb)


def _er_body(e_ref, dm_ref, o_ref):
    o_ref[...] = (
        jnp.dot(e_ref[...], dm_ref[0:16, :], preferred_element_type=F32)
        + dm_ref[16:17, :]
    )


def _er(e_raw, dm):
    d = e_raw.shape[1]
    return pl.pallas_call(
        _er_body,
        grid=(E // BE,),
        in_specs=[
            pl.BlockSpec((BE, d), lambda i: (i, 0)),
            pl.BlockSpec((DT, H), lambda i: (0, 0)),
        ],
        out_specs=pl.BlockSpec((BE, H), lambda i: (i, 0)),
        out_shape=jax.ShapeDtypeStruct((E, H), F32),
    )(e_raw, dm)


def _hupd_body(agg_ref, qr_ref, deg_ref, p4_ref, w_ref, hb_ref, o_ref):
    deg = deg_ref[0, :, 0:1] + deg_ref[1, :, 0:1]
    x = agg_ref[...] + deg * qr_ref[...]
    o_ref[...] = (
        p4_ref[...]
        + jnp.dot(x, w_ref[...], preferred_element_type=F32)
        + hb_ref[...]
    )


def _hupd(agg, qr, deg2, p4, wh_m, hupd_b):
    return pl.pallas_call(
        _hupd_body,
        grid=(N // BN,),
        in_specs=[
            pl.BlockSpec((BN, H), lambda i: (i, 0)),
            pl.BlockSpec((BN, H), lambda i: (i, 0)),
            pl.BlockSpec((2, BN, 128), lambda i: (0, i, 0)),
            pl.BlockSpec((BN, H), lambda i: (i, 0)),
            pl.BlockSpec((H, H), lambda i: (0, 0)),
            pl.BlockSpec((1, H), lambda i: (0, 0)),
        ],
        out_specs=pl.BlockSpec((BN, H), lambda i: (i, 0)),
        out_shape=jax.ShapeDtypeStruct((N, H), F32),
    )(agg, qr, deg2, p4, wh_m, hupd_b)


GC = 40
SC_C = 80
CCH = 128
NP = 10240


def _make_sc_scatter():
    n_per = E // NS
    n_it = n_per // SC_C
    rz = NP // NS
    zb = 80
    n_rows_it = rz // zb

    @functools.partial(
        pl.kernel,
        out_type=jax.ShapeDtypeStruct((NP, H), F32),
        mesh=_mesh(),
        scratch_types=[
            pltpu.VMEM((SC_C,), jnp.int32),
            pltpu.VMEM((SC_C,), jnp.int32),
            pltpu.VMEM((SC_C, CCH), F32),
            pltpu.VMEM((SC_C, CCH), F32),
            pltpu.VMEM((SC_C,), jnp.int32),
            pltpu.VMEM((SC_C,), jnp.int32),
            pltpu.VMEM((SC_C, CCH), F32),
            pltpu.VMEM((SC_C, CCH), F32),
            pltpu.VMEM_SHARED((NP, CCH), F32),
            pltpu.SemaphoreType.DMA,
            pltpu.SemaphoreType.DMA,
            pltpu.SemaphoreType.DMA,
            pltpu.SemaphoreType.DMA,
        ],
    )
    def k(p0, p1, p2, p3, er, send, rec, zeros_hbm, out,
          sidx_a, ridx_a, val_a, gat_a, sidx_b, ridx_b, val_b, gat_b,
          acc_sh, semg_a, seme_a, semg_b, seme_b):
        cid = lax.axis_index("c")
        sid = lax.axis_index("s")
        e_base = sid * n_per
        r_base = sid * rz
        tables = (p0, p1, p2, p3)
        buf_a = (sidx_a, ridx_a, val_a, gat_a, semg_a, seme_a)
        buf_b = (sidx_b, ridx_b, val_b, gat_b, semg_b, seme_b)

        for chunk_k in range(4):
            @pl.when(cid == chunk_k // 2)
            def _pass(chunk_k=chunk_k):
                ccol = chunk_k * CCH
                table = tables[chunk_k]

                def load(buf, i):
                    sidx, ridx, val, gat, semg, seme = buf
                    off = e_base + i * SC_C
                    pltpu.sync_copy(send.at[pl.ds(off, SC_C)], sidx)
                    pltpu.async_copy(table.at[sidx], gat, semg)
                    pltpu.sync_copy(rec.at[pl.ds(off, SC_C)], ridx)
                    pltpu.async_copy(
                        er.at[pl.ds(off, SC_C), pl.ds(ccol, CCH)], val, seme
                    )

                def drain(buf, i):
                    sidx, ridx, val, gat, semg, seme = buf
                    off = e_base + i * SC_C
                    pltpu.make_async_copy(
                        er.at[pl.ds(off, SC_C), pl.ds(ccol, CCH)], val, seme
                    ).wait()
                    pltpu.sync_copy(val, acc_sh.at[ridx], add=True)
                    pltpu.make_async_copy(table.at[sidx], gat, semg).wait()
                    pltpu.sync_copy(gat, acc_sh.at[ridx], add=True)

                pltpu.sync_copy(zeros_hbm.at[pl.ds(0, zb)], val_a)
                for t in range(n_rows_it):
                    pltpu.sync_copy(
                        val_a, acc_sh.at[pl.ds(r_base + t * zb, zb)]
                    )
                plsc.subcore_barrier()

                load(buf_a, 0)

                def body(i, carry):
                    @pl.when(i % 2 == 0)
                    def _even():
                        @pl.when(i + 1 < n_it)
                        def _pf():
                            load(buf_b, i + 1)
                        drain(buf_a, i)

                    @pl.when(i % 2 == 1)
                    def _odd():
                        @pl.when(i + 1 < n_it)
                        def _pf():
                            load(buf_a, i + 1)
                        drain(buf_b, i)

                    return carry

                lax.fori_loop(0, n_it, body, 0)
                plsc.subcore_barrier()

                for t in range(n_rows_it):
                    r0 = r_base + t * zb
                    pltpu.sync_copy(acc_sh.at[pl.ds(r0, zb)], val_a)
                    pltpu.sync_copy(
                        val_a, out.at[pl.ds(r0, zb), pl.ds(ccol, CCH)]
                    )
                plsc.subcore_barrier()

    return k


_sc_scatter = _make_sc_scatter()


def _make_sc_deg():
    n_per = E // NW
    n_it = n_per // GC
    rz = NP // NS
    zb = 128
    n_rows_it = rz // zb

    @functools.partial(
        pl.kernel,
        out_type=jax.ShapeDtypeStruct((NC * NP, CCH), F32),
        mesh=_mesh(),
        scratch_types=[
            pltpu.VMEM((GC,), jnp.int32),
            pltpu.VMEM((GC, CCH), F32),
            pltpu.VMEM((zb, CCH), F32),
            pltpu.VMEM_SHARED((NP, CCH), F32),
        ],
    )
    def k(rec, ones_hbm, zeros_hbm, out, idx_v, ones_v, rbuf_v, acc_sh):
        cid = lax.axis_index("c")
        sid = lax.axis_index("s")
        wid = sid * NC + cid
        e_base = wid * n_per
        r_base = sid * rz

        pltpu.sync_copy(zeros_hbm, rbuf_v)
        for t in range(n_rows_it):
            pltpu.sync_copy(rbuf_v, acc_sh.at[pl.ds(r_base + t * zb, zb)])
        pltpu.sync_copy(ones_hbm, ones_v)
        plsc.subcore_barrier()

        def body(i, carry):
            off = e_base + i * GC
            pltpu.sync_copy(rec.at[pl.ds(off, GC)], idx_v)
            pltpu.sync_copy(ones_v, acc_sh.at[idx_v], add=True)
            return carry

        lax.fori_loop(0, n_it, body, 0)
        plsc.subcore_barrier()

        for t in range(n_rows_it):
            r0 = r_base + t * zb
            pltpu.sync_copy(acc_sh.at[pl.ds(r0, zb)], rbuf_v)
            pltpu.sync_copy(rbuf_v, out.at[pl.ds(cid * NP + r0, zb)])

    return k


_sc_deg = _make_sc_deg()


def kernel(h, e, edge_index, embed_W, embed_b, edge_embed_W, edge_embed_b,
           msg_W, msg_b, hupd_W, hupd_b, eupd_W, eupd_b):
    send = edge_index[0]
    rec = edge_index[1]

    zeros_hbm = jnp.zeros((128, CCH), F32)
    ones_hbm = jnp.ones((GC, CCH), F32)

    deg2 = _sc_deg(rec, ones_hbm, zeros_hbm).reshape(NC, NP, CCH)

    dt = jnp.zeros((DT, H), F32)
    dt = dt.at[0:16, :].set(edge_embed_W).at[16, :].set(edge_embed_b)

    ers = []
    for l in range(L):
        dm, dt = _composite(
            dt, msg_W[l, 2 * H:], eupd_W[l, 2 * H:],
            msg_b[l].reshape(1, H), eupd_b[l].reshape(1, H),
        )
        ers.append(_er(e, dm))

    def wslices(l):
        return (msg_W[l, :H], msg_W[l, H:2 * H], msg_W[l, 2 * H:],
                eupd_W[l, :H], eupd_W[l, H:2 * H], eupd_W[l, 2 * H:],
                hupd_W[l, :H], hupd_W[l, H:])

    Wm_s, Wm_r, Wm_e, We_s, We_r, We_e, Wh_h, Wh_m = wslices(0)
    wcat0 = jnp.concatenate([Wm_s, Wm_r, Wh_h, We_s, We_r], axis=1)
    pm0, pm1, pm2, pm3, qr, p4, u, v = _node_first(
        h, embed_W, embed_b.reshape(1, H), wcat0
    )
    agg = _sc_scatter(pm0, pm1, pm2, pm3, ers[0], send, rec, zeros_hbm)

    Wm_s1, Wm_r1, Wm_e1, We_s1, We_r1, We_e1, Wh_h1, Wh_m1 = wslices(1)
    wcat1 = jnp.concatenate([Wm_s1, Wm_r1, Wh_h1, We_s1, We_r1], axis=1)
    pm0, pm1, pm2, pm3, qr1, p41, u2, v2 = _hupd_node(
        agg, qr, deg2, p4, Wh_m, hupd_b[0].reshape(1, H),
        u, v, wcat1, Wm_e1, We_e1,
    )
    agg1 = _sc_scatter(pm0, pm1, pm2, pm3, ers[1], send, rec, zeros_hbm)

    Wm_s2, Wm_r2, Wm_e2, We_s2, We_r2, We_e2, Wh_h2, Wh_m2 = wslices(2)
    wcat2 = jnp.concatenate([Wm_s2, Wm_r2, Wh_h2], axis=1)
    pm0, pm1, pm2, pm3, qr2, p42 = _hupd_node(
        agg1, qr1, deg2, p41, Wh_m1, hupd_b[1].reshape(1, H),
        u2, v2, wcat2, Wm_e2,
    )
    agg2 = _sc_scatter(pm0, pm1, pm2, pm3, ers[2], send, rec, zeros_hbm)

    return _hupd(agg2, qr2, deg2, p42, Wh_m2, hupd_b[2].reshape(1, H))


# speedup vs baseline: 4.1297x; 1.0049x over previous
import functools

import jax
import jax.numpy as jnp
from jax import lax
from jax.experimental import pallas as pl
from jax.experimental.pallas import tpu as pltpu
from jax.experimental.pallas import tpu_sc as plsc

N = 10000
E = 160000
H = 512
L = 3
NC = 2
NS = 16
NW = NC * NS

F32 = jnp.float32


def _mesh():
    return plsc.VectorSubcoreMesh(
        core_axis_name="c", subcore_axis_name="s", num_cores=NC, num_subcores=NS
    )


BN = 1000
BE = 3200
DT = 24


_W_FULL = [128, 128, 128, 128, H, H, H, H]
_W_SHORT = [128, 128, 128, 128, H, H]


def _project(r, u, v, wme_ref, wee_ref, out_refs):
    qs = r[:, 0:H] + jnp.dot(u, wme_ref[...], preferred_element_type=F32)
    qr = r[:, H:2 * H] + jnp.dot(v, wme_ref[...], preferred_element_type=F32)
    for j in range(4):
        out_refs[j][...] = qs[:, j * 128:(j + 1) * 128]
    out_refs[4][...] = qr
    out_refs[5][...] = r[:, 2 * H:3 * H]
    if wee_ref is not None:
        out_refs[6][...] = r[:, 3 * H:4 * H] + jnp.dot(
            u, wee_ref[...], preferred_element_type=F32
        )
        out_refs[7][...] = r[:, 4 * H:5 * H] + jnp.dot(
            v, wee_ref[...], preferred_element_type=F32
        )


def _node_first_body(x_ref, ew_ref, eb_ref, w_ref, *out_refs):
    h0 = jnp.dot(x_ref[...], ew_ref[...], preferred_element_type=F32)
    h0 = h0 + eb_ref[...]
    r = jnp.dot(h0, w_ref[...], preferred_element_type=F32)
    off = 0
    for o_ref, w in zip(out_refs, _W_FULL):
        o_ref[...] = r[:, off:off + w]
        off += w


def _node_first(x, embw, embb, wcat):
    d = x.shape[1]
    return pl.pallas_call(
        _node_first_body,
        grid=(N // BN,),
        in_specs=[
            pl.BlockSpec((BN, d), lambda i: (i, 0)),
            pl.BlockSpec((d, H), lambda i: (0, 0)),
            pl.BlockSpec((1, H), lambda i: (0, 0)),
            pl.BlockSpec((H, 5 * H), lambda i: (0, 0)),
        ],
        out_specs=[pl.BlockSpec((BN, w), lambda i: (i, 0)) for w in _W_FULL],
        out_shape=[jax.ShapeDtypeStruct((N, w), F32) for w in _W_FULL],
    )(x, embw, embb, wcat)


def _hupd_node_body(agg_ref, qr_ref, deg_ref, p4_ref, whm_ref, hb_ref,
                    u_ref, v_ref, w_ref, wme_ref, wee_ref, *out_refs):
    deg = deg_ref[0, :, 0:1] + deg_ref[1, :, 0:1]
    agg = jnp.concatenate([agg_ref[j] for j in range(4)], axis=1)
    x = agg + deg * qr_ref[...]
    h = (
        p4_ref[...]
        + jnp.dot(x, whm_ref[...], preferred_element_type=F32)
        + hb_ref[...]
    )
    r = jnp.dot(h, w_ref[...], preferred_element_type=F32)
    _project(r, u_ref[...], v_ref[...], wme_ref, wee_ref, out_refs)


def _hupd_node_last_body(agg_ref, qr_ref, deg_ref, p4_ref, whm_ref, hb_ref,
                         u_ref, v_ref, w_ref, wme_ref, *out_refs):
    _hupd_node_body(agg_ref, qr_ref, deg_ref, p4_ref, whm_ref, hb_ref,
                    u_ref, v_ref, w_ref, wme_ref, None, *out_refs)


def _hupd_node(agg, qr, deg2, p4, whm, hb, u, v, wcat, wme, wee=None):
    widths = _W_FULL if wee is not None else _W_SHORT
    nw = wcat.shape[1]
    in_specs = [
        pl.BlockSpec((4, BN, 128), lambda i: (0, i, 0)),
        pl.BlockSpec((BN, H), lambda i: (i, 0)),
        pl.BlockSpec((2, BN, 128), lambda i: (0, i, 0)),
        pl.BlockSpec((BN, H), lambda i: (i, 0)),
        pl.BlockSpec((H, H), lambda i: (0, 0)),
        pl.BlockSpec((1, H), lambda i: (0, 0)),
        pl.BlockSpec((BN, H), lambda i: (i, 0)),
        pl.BlockSpec((BN, H), lambda i: (i, 0)),
        pl.BlockSpec((H, nw), lambda i: (0, 0)),
        pl.BlockSpec((H, H), lambda i: (0, 0)),
    ]
    args = [agg, qr, deg2, p4, whm, hb, u, v, wcat, wme]
    body = _hupd_node_last_body
    if wee is not None:
        in_specs.append(pl.BlockSpec((H, H), lambda i: (0, 0)))
        args.append(wee)
        body = _hupd_node_body
    return pl.pallas_call(
        body,
        grid=(N // BN,),
        in_specs=in_specs,
        out_specs=[pl.BlockSpec((BN, w), lambda i: (i, 0)) for w in widths],
        out_shape=[jax.ShapeDtypeStruct((N, w), F32) for w in widths],
    )(*args)


def _composite_body(dt_ref, wme_ref, wee_ref, mb_ref, eb_ref, dm_ref, dn_ref):
    dt = dt_ref[...]
    rows = lax.broadcasted_iota(jnp.int32, (DT, H), 0)
    dm = jnp.dot(dt, wme_ref[...], preferred_element_type=F32)
    dm_ref[...] = dm + jnp.where(rows == 16, mb_ref[...], 0.0)
    dn = jnp.dot(dt, wee_ref[...], preferred_element_type=F32)
    dn_ref[...] = dn + jnp.where(rows == 16, eb_ref[...], 0.0)


def _composite(dt, wme, wee, mb, eb):
    return pl.pallas_call(
        _composite_body,
        grid=(1,),
        in_specs=[
            pl.BlockSpec((DT, H), lambda i: (0, 0)),
            pl.BlockSpec((H, H), lambda i: (0, 0)),
            pl.BlockSpec((H, H), lambda i: (0, 0)),
            pl.BlockSpec((1, H), lambda i: (0, 0)),
            pl.BlockSpec((1, H), lambda i: (0, 0)),
        ],
        out_specs=[
            pl.BlockSpec((DT, H), lambda i: (0, 0)),
            pl.BlockSpec((DT, H), lambda i: (0, 0)),
        ],
        out_shape=[
            jax.ShapeDtypeStruct((DT, H), F32),
            jax.ShapeDtypeStruct((DT, H), F32),
        ],
    )(dt, wme, wee, mb, eb)


def _er_body(e_ref, dm_ref, *o_refs):
    er = (
        jnp.dot(e_ref[...], dm_ref[0:16, :], preferred_element_type=F32)
        + dm_ref[16:17, :]
    )
    for j in range(4):
        o_refs[j][...] = er[:, j * 128:(j + 1) * 128]


def _er(e_raw, dm):
    d = e_raw.shape[1]
    return pl.pallas_call(
        _er_body,
        grid=(E // BE,),
        in_specs=[
            pl.BlockSpec((BE, d), lambda i: (i, 0)),
            pl.BlockSpec((DT, H), lambda i: (0, 0)),
        ],
        out_specs=[pl.BlockSpec((BE, 128), lambda i: (i, 0))] * 4,
        out_shape=[jax.ShapeDtypeStruct((E, 128), F32)] * 4,
    )(e_raw, dm)


def _hupd_body(agg_ref, qr_ref, deg_ref, p4_ref, w_ref, hb_ref, o_ref):
    deg = deg_ref[0, :, 0:1] + deg_ref[1, :, 0:1]
    agg = jnp.concatenate([agg_ref[j] for j in range(4)], axis=1)
    x = agg + deg * qr_ref[...]
    o_ref[...] = (
        p4_ref[...]
        + jnp.dot(x, w_ref[...], preferred_element_type=F32)
        + hb_ref[...]
    )


def _hupd(agg, qr, deg2, p4, wh_m, hupd_b):
    return pl.pallas_call(
        _hupd_body,
        grid=(N // BN,),
        in_specs=[
            pl.BlockSpec((4, BN, 128), lambda i: (0, i, 0)),
            pl.BlockSpec((BN, H), lambda i: (i, 0)),
            pl.BlockSpec((2, BN, 128), lambda i: (0, i, 0)),
            pl.BlockSpec((BN, H), lambda i: (i, 0)),
            pl.BlockSpec((H, H), lambda i: (0, 0)),
            pl.BlockSpec((1, H), lambda i: (0, 0)),
        ],
        out_specs=pl.BlockSpec((BN, H), lambda i: (i, 0)),
        out_shape=jax.ShapeDtypeStruct((N, H), F32),
    )(agg, qr, deg2, p4, wh_m, hupd_b)


GC = 40
SC_C = 80
CCH = 128
NP = 10240


def _make_sc_scatter():
    n_per = E // NS
    n_it = n_per // SC_C
    rz = NP // NS
    zb = 80
    n_rows_it = rz // zb

    @functools.partial(
        pl.kernel,
        out_type=jax.ShapeDtypeStruct((4 * NP, CCH), F32),
        mesh=_mesh(),
        scratch_types=[
            pltpu.VMEM((SC_C,), jnp.int32),
            pltpu.VMEM((SC_C,), jnp.int32),
            pltpu.VMEM((SC_C, CCH), F32),
            pltpu.VMEM((SC_C, CCH), F32),
            pltpu.VMEM((SC_C,), jnp.int32),
            pltpu.VMEM((SC_C,), jnp.int32),
            pltpu.VMEM((SC_C, CCH), F32),
            pltpu.VMEM((SC_C, CCH), F32),
            pltpu.VMEM_SHARED((NP, CCH), F32),
            pltpu.SemaphoreType.DMA,
            pltpu.SemaphoreType.DMA,
            pltpu.SemaphoreType.DMA,
            pltpu.SemaphoreType.DMA,
        ],
    )
    def k(p0, p1, p2, p3, er0, er1, er2, er3, send, rec, zeros_hbm, out,
          sidx_a, ridx_a, val_a, gat_a, sidx_b, ridx_b, val_b, gat_b,
          acc_sh, semg_a, seme_a, semg_b, seme_b):
        cid = lax.axis_index("c")
        sid = lax.axis_index("s")
        e_base = sid * n_per
        r_base = sid * rz
        tables = (p0, p1, p2, p3)
        ers = (er0, er1, er2, er3)
        buf_a = (sidx_a, ridx_a, val_a, gat_a, semg_a, seme_a)
        buf_b = (sidx_b, ridx_b, val_b, gat_b, semg_b, seme_b)

        for chunk_k in range(4):
            @pl.when(cid == chunk_k // 2)
            def _pass(chunk_k=chunk_k):
                table = tables[chunk_k]
                erc = ers[chunk_k]

                def load(buf, i):
                    sidx, ridx, val, gat, semg, seme = buf
                    off = e_base + i * SC_C
                    pltpu.sync_copy(send.at[pl.ds(off, SC_C)], sidx)
                    pltpu.async_copy(table.at[sidx], gat, semg)
                    pltpu.sync_copy(rec.at[pl.ds(off, SC_C)], ridx)
                    pltpu.async_copy(erc.at[pl.ds(off, SC_C)], val, seme)

                def drain(buf, i):
                    sidx, ridx, val, gat, semg, seme = buf
                    off = e_base + i * SC_C
                    pltpu.make_async_copy(
                        erc.at[pl.ds(off, SC_C)], val, seme
                    ).wait()
                    pltpu.sync_copy(val, acc_sh.at[ridx], add=True)
                    pltpu.make_async_copy(table.at[sidx], gat, semg).wait()
                    pltpu.sync_copy(gat, acc_sh.at[ridx], add=True)

                pltpu.sync_copy(zeros_hbm.at[pl.ds(0, zb)], val_a)
                for t in range(n_rows_it):
                    pltpu.sync_copy(
                        val_a, acc_sh.at[pl.ds(r_base + t * zb, zb)]
                    )
                plsc.subcore_barrier()

                load(buf_a, 0)

                def body(i, carry):
                    @pl.when(i % 2 == 0)
                    def _even():
                        @pl.when(i + 1 < n_it)
                        def _pf():
                            load(buf_b, i + 1)
                        drain(buf_a, i)

                    @pl.when(i % 2 == 1)
                    def _odd():
                        @pl.when(i + 1 < n_it)
                        def _pf():
                            load(buf_a, i + 1)
                        drain(buf_b, i)

                    return carry

                lax.fori_loop(0, n_it, body, 0)
                plsc.subcore_barrier()

                for t in range(n_rows_it):
                    r0 = r_base + t * zb
                    pltpu.sync_copy(acc_sh.at[pl.ds(r0, zb)], val_a)
                    pltpu.sync_copy(
                        val_a, out.at[pl.ds(chunk_k * NP + r0, zb)]
                    )
                plsc.subcore_barrier()

    return k


_sc_scatter = _make_sc_scatter()


def _make_sc_deg():
    n_per = E // NW
    n_it = n_per // GC
    rz = NP // NS
    zb = 128
    n_rows_it = rz // zb

    @functools.partial(
        pl.kernel,
        out_type=jax.ShapeDtypeStruct((NC * NP, CCH), F32),
        mesh=_mesh(),
        scratch_types=[
            pltpu.VMEM((GC,), jnp.int32),
            pltpu.VMEM((GC, CCH), F32),
            pltpu.VMEM((zb, CCH), F32),
            pltpu.VMEM_SHARED((NP, CCH), F32),
        ],
    )
    def k(rec, ones_hbm, zeros_hbm, out, idx_v, ones_v, rbuf_v, acc_sh):
        cid = lax.axis_index("c")
        sid = lax.axis_index("s")
        wid = sid * NC + cid
        e_base = wid * n_per
        r_base = sid * rz

        pltpu.sync_copy(zeros_hbm, rbuf_v)
        for t in range(n_rows_it):
            pltpu.sync_copy(rbuf_v, acc_sh.at[pl.ds(r_base + t * zb, zb)])
        pltpu.sync_copy(ones_hbm, ones_v)
        plsc.subcore_barrier()

        def body(i, carry):
            off = e_base + i * GC
            pltpu.sync_copy(rec.at[pl.ds(off, GC)], idx_v)
            pltpu.sync_copy(ones_v, acc_sh.at[idx_v], add=True)
            return carry

        lax.fori_loop(0, n_it, body, 0)
        plsc.subcore_barrier()

        for t in range(n_rows_it):
            r0 = r_base + t * zb
            pltpu.sync_copy(acc_sh.at[pl.ds(r0, zb)], rbuf_v)
            pltpu.sync_copy(rbuf_v, out.at[pl.ds(cid * NP + r0, zb)])

    return k


_sc_deg = _make_sc_deg()


def kernel(h, e, edge_index, embed_W, embed_b, edge_embed_W, edge_embed_b,
           msg_W, msg_b, hupd_W, hupd_b, eupd_W, eupd_b):
    send = edge_index[0]
    rec = edge_index[1]

    zeros_hbm = jnp.zeros((128, CCH), F32)
    ones_hbm = jnp.ones((GC, CCH), F32)

    deg2 = _sc_deg(rec, ones_hbm, zeros_hbm).reshape(NC, NP, CCH)

    dt = jnp.zeros((DT, H), F32)
    dt = dt.at[0:16, :].set(edge_embed_W).at[16, :].set(edge_embed_b)

    ers = []
    for l in range(L):
        dm, dt = _composite(
            dt, msg_W[l, 2 * H:], eupd_W[l, 2 * H:],
            msg_b[l].reshape(1, H), eupd_b[l].reshape(1, H),
        )
        ers.append(_er(e, dm))

    def wslices(l):
        return (msg_W[l, :H], msg_W[l, H:2 * H], msg_W[l, 2 * H:],
                eupd_W[l, :H], eupd_W[l, H:2 * H], eupd_W[l, 2 * H:],
                hupd_W[l, :H], hupd_W[l, H:])

    Wm_s, Wm_r, Wm_e, We_s, We_r, We_e, Wh_h, Wh_m = wslices(0)
    wcat0 = jnp.concatenate([Wm_s, Wm_r, Wh_h, We_s, We_r], axis=1)
    pm0, pm1, pm2, pm3, qr, p4, u, v = _node_first(
        h, embed_W, embed_b.reshape(1, H), wcat0
    )
    agg = _sc_scatter(pm0, pm1, pm2, pm3, *ers[0], send, rec,
                      zeros_hbm).reshape(4, NP, CCH)

    Wm_s1, Wm_r1, Wm_e1, We_s1, We_r1, We_e1, Wh_h1, Wh_m1 = wslices(1)
    wcat1 = jnp.concatenate([Wm_s1, Wm_r1, Wh_h1, We_s1, We_r1], axis=1)
    pm0, pm1, pm2, pm3, qr1, p41, u2, v2 = _hupd_node(
        agg, qr, deg2, p4, Wh_m, hupd_b[0].reshape(1, H),
        u, v, wcat1, Wm_e1, We_e1,
    )
    agg1 = _sc_scatter(pm0, pm1, pm2, pm3, *ers[1], send, rec,
                       zeros_hbm).reshape(4, NP, CCH)

    Wm_s2, Wm_r2, Wm_e2, We_s2, We_r2, We_e2, Wh_h2, Wh_m2 = wslices(2)
    wcat2 = jnp.concatenate([Wm_s2, Wm_r2, Wh_h2], axis=1)
    pm0, pm1, pm2, pm3, qr2, p42 = _hupd_node(
        agg1, qr1, deg2, p41, Wh_m1, hupd_b[1].reshape(1, H),
        u2, v2, wcat2, Wm_e2,
    )
    agg2 = _sc_scatter(pm0, pm1, pm2, pm3, *ers[2], send, rec,
                       zeros_hbm).reshape(4, NP, CCH)

    return _hupd(agg2, qr2, deg2, p42, Wh_m2, hupd_b[2].reshape(1, H))
